# Initial kernel scaffold; baseline (speedup 1.0000x reference)
import jax
import jax.numpy as jnp
from jax.experimental import pallas as pl


def kernel(x, W, b, a):
    raise NotImplementedError("write your pallas kernel here")


# speedup vs baseline: 19.5224x; 19.5224x over previous
import jax
import jax.numpy as jnp
from jax.experimental import pallas as pl
from jax.experimental.pallas import tpu as pltpu

BATCH = 64
CIN = 128
C = 256
V = 256
K = 16


def _s_kernel(x_ref, w_ref, b_ref, a1_ref, a2_ref, out_ref):
    x_b = x_ref[0]
    pre = jnp.dot(w_ref[...], x_b, preferred_element_type=jnp.float32)
    pre = pre + b_ref[...]
    g = jax.lax.dot_general(pre, pre, (((0,), (0,)), ((), ())),
                            preferred_element_type=jnp.float32)
    d = jnp.sum(pre * pre, axis=0, keepdims=True)
    inner = -2.0 * g
    p = (-d - inner) - jnp.transpose(d)
    t1 = jnp.dot(a1_ref[...], pre, preferred_element_type=jnp.float32)
    t2 = jnp.dot(a2_ref[...], pre, preferred_element_type=jnp.float32)
    iota_w = jax.lax.broadcasted_iota(jnp.int32, (V, V), 1)
    iota_v = jax.lax.broadcasted_iota(jnp.int32, (V, V), 0)
    gsel_base = 16 * jax.lax.rem(iota_v, 16)
    t1_bcast = jnp.broadcast_to(t1, (V, V))
    t2_bcast = jnp.broadcast_to(t2, (V, V))
    col_iota = jax.lax.broadcasted_iota(jnp.int32, (V, K), 1)
    s_acc = jnp.zeros((V, K), jnp.float32)
    for j in range(K):
        m = jnp.max(p, axis=1, keepdims=True)
        is_max = p == m
        idx = jnp.min(jnp.where(is_max, iota_w, V), axis=1, keepdims=True)
        onehot = iota_w == idx
        s_col = jnp.sum(jnp.where(onehot, t2_bcast, 0.0), axis=1, keepdims=True)
        oh1 = iota_w == (gsel_base + j)
        t1_col = jnp.sum(jnp.where(oh1, t1_bcast, 0.0), axis=1, keepdims=True)
        s_acc = jnp.where(col_iota == j, t1_col + s_col, s_acc)
        if j + 1 < K:
            p = jnp.where(onehot, -jnp.inf, p)
    out_ref[0] = s_acc


def _softmax0_kernel(s_ref, out_ref):
    s = s_ref[...]
    m = jnp.max(s, axis=0, keepdims=True)
    e = jnp.exp(s - m)
    out_ref[...] = e / jnp.sum(e, axis=0, keepdims=True)


def kernel(x, W, b, a):
    a1 = a[:C, 0].reshape(1, C)
    a2 = a[C:, 0].reshape(1, C)
    b2 = b.reshape(C, 1)
    s = pl.pallas_call(
        _s_kernel,
        grid=(BATCH,),
        in_specs=[
            pl.BlockSpec((1, CIN, V), lambda i: (i, 0, 0)),
            pl.BlockSpec((C, CIN), lambda i: (0, 0)),
            pl.BlockSpec((C, 1), lambda i: (0, 0)),
            pl.BlockSpec((1, C), lambda i: (0, 0)),
            pl.BlockSpec((1, C), lambda i: (0, 0)),
        ],
        out_specs=pl.BlockSpec((1, V, K), lambda i: (i, 0, 0)),
        out_shape=jax.ShapeDtypeStruct((BATCH, V, K), jnp.float32),
    )(x, W, b2, a1, a2)
    h = pl.pallas_call(
        _softmax0_kernel,
        out_shape=jax.ShapeDtypeStruct((BATCH, V * K), jnp.float32),
    )(s.reshape(BATCH, V * K))
    return h.reshape(BATCH, V, K)


# speedup vs baseline: 27.9005x; 1.4292x over previous
import jax
import jax.numpy as jnp
from jax.experimental import pallas as pl
from jax.experimental.pallas import tpu as pltpu

BATCH = 64
CIN = 128
C = 256
V = 256
K = 16


def _s_kernel(x_ref, w_ref, b_ref, a1_ref, a2_ref, out_ref, t1_ref):
    x_b = x_ref[0]
    pre = jnp.dot(w_ref[...], x_b, preferred_element_type=jnp.float32)
    pre = pre + b_ref[...]
    g = jax.lax.dot_general(pre, pre, (((0,), (0,)), ((), ())),
                            preferred_element_type=jnp.float32)
    d = jnp.sum(pre * pre, axis=0, keepdims=True)
    inner = -2.0 * g
    p = (-d - inner) - jnp.transpose(d)
    t1 = jnp.dot(a1_ref[...], pre, preferred_element_type=jnp.float32)
    t2 = jnp.dot(a2_ref[...], pre, preferred_element_type=jnp.float32)
    t1_ref[0] = t1
    rev_iota = (jnp.float32(V)
                - jax.lax.broadcasted_iota(jnp.int32, (V, V), 1).astype(jnp.float32))
    t2_bcast = jnp.broadcast_to(t2, (V, V))
    neg_inf = jnp.float32(-jnp.inf)
    for j in range(K):
        m = jnp.max(p, axis=1, keepdims=True)
        z = jnp.where(p == m, rev_iota, 0.0)
        mx = jnp.max(z, axis=1, keepdims=True)
        onehot = z == mx
        s_col = jnp.max(jnp.where(onehot, t2_bcast, neg_inf),
                        axis=1, keepdims=True)
        out_ref[0, :, j:j + 1] = s_col
        if j + 1 < K:
            p = jnp.where(onehot, neg_inf, p)


def _softmax0_kernel(s2_ref, t1_ref, out_ref):
    t1 = t1_ref[...]
    s = s2_ref[...] + jnp.concatenate([t1] * K, axis=1)
    m = jnp.max(s, axis=0, keepdims=True)
    e = jnp.exp(s - m)
    out_ref[...] = e / jnp.sum(e, axis=0, keepdims=True)


def kernel(x, W, b, a):
    a1 = a[:C, 0].reshape(1, C)
    a2 = a[C:, 0].reshape(1, C)
    b2 = b.reshape(C, 1)
    s2, t1 = pl.pallas_call(
        _s_kernel,
        grid=(BATCH,),
        in_specs=[
            pl.BlockSpec((1, CIN, V), lambda i: (i, 0, 0)),
            pl.BlockSpec((C, CIN), lambda i: (0, 0)),
            pl.BlockSpec((C, 1), lambda i: (0, 0)),
            pl.BlockSpec((1, C), lambda i: (0, 0)),
            pl.BlockSpec((1, C), lambda i: (0, 0)),
        ],
        out_specs=[
            pl.BlockSpec((1, V, K), lambda i: (i, 0, 0)),
            pl.BlockSpec((1, 1, V), lambda i: (i, 0, 0)),
        ],
        out_shape=[
            jax.ShapeDtypeStruct((BATCH, V, K), jnp.float32),
            jax.ShapeDtypeStruct((BATCH, 1, V), jnp.float32),
        ],
    )(x, W, b2, a1, a2)
    h = pl.pallas_call(
        _softmax0_kernel,
        out_shape=jax.ShapeDtypeStruct((BATCH, V * K), jnp.float32),
    )(s2.reshape(BATCH, V * K), t1.reshape(BATCH, V))
    return h.reshape(BATCH, V, K)


# speedup vs baseline: 44.1288x; 1.5816x over previous
import jax
import jax.numpy as jnp
from jax.experimental import pallas as pl
from jax.experimental.pallas import tpu as pltpu

BATCH = 64
CIN = 128
C = 256
V = 256
K = 16

GROUP = 1


def _s_kernel(x_ref, w_ref, b_ref, a1_ref, a2_ref, out_ref):
    iota_w0 = jax.lax.broadcasted_iota(jnp.int32, (V, V), 0)
    rev_iota = (jnp.float32(V) - iota_w0.astype(jnp.float32))
    bsel = (iota_w0 // 16
            == jax.lax.rem(jax.lax.broadcasted_iota(jnp.int32, (V, V), 1), 16)
            ).astype(jnp.float32)
    jota = jax.lax.broadcasted_iota(jnp.int32, (K, C), 0)
    wmod = jax.lax.rem(jax.lax.broadcasted_iota(jnp.int32, (K, C), 1), 16)
    neg_inf = jnp.float32(-jnp.inf)
    for gi in range(GROUP):
        x_b = x_ref[gi]
        pre = jnp.dot(w_ref[...], x_b, preferred_element_type=jnp.float32)
        pre = pre + b_ref[...]
        g = jax.lax.dot_general(pre, pre, (((0,), (0,)), ((), ())),
                                preferred_element_type=jnp.float32)
        d = jnp.sum(pre * pre, axis=0, keepdims=True)
        inner = -2.0 * g
        p = (-jnp.transpose(d) - inner) - d
        t1 = jnp.dot(a1_ref[...], pre, preferred_element_type=jnp.float32)
        t2c = jax.lax.dot_general(pre, a2_ref[...], (((0,), (0,)), ((), ())),
                                  preferred_element_type=jnp.float32)
        t2_bcast = jnp.broadcast_to(t2c, (V, V))
        u = jnp.where(wmod == jota, jnp.broadcast_to(t1, (K, C)), 0.0)
        t1p = jnp.dot(u, bsel, preferred_element_type=jnp.float32)
        for j in range(K):
            m = jnp.max(p, axis=0, keepdims=True)
            z = jnp.where(p == m, rev_iota, 0.0)
            mx = jnp.max(z, axis=0, keepdims=True)
            onehot = z == mx
            s_row = jnp.max(jnp.where(onehot, t2_bcast, neg_inf),
                            axis=0, keepdims=True)
            out_ref[gi, j:j + 1, :] = s_row + t1p[j:j + 1, :]
            if j + 1 < K:
                p = jnp.where(onehot, neg_inf, p)


def _softmax0_kernel(s_ref, out_ref):
    s = s_ref[...]
    m = jnp.max(s, axis=0, keepdims=True)
    e = jnp.exp(s - m)
    out_ref[...] = e / jnp.sum(e, axis=0, keepdims=True)


def kernel(x, W, b, a):
    a1 = a[:C, 0].reshape(1, C)
    a2 = a[C:, 0].reshape(C, 1)
    b2 = b.reshape(C, 1)
    s = pl.pallas_call(
        _s_kernel,
        grid=(BATCH // GROUP,),
        in_specs=[
            pl.BlockSpec((GROUP, CIN, V), lambda i: (i, 0, 0)),
            pl.BlockSpec((C, CIN), lambda i: (0, 0)),
            pl.BlockSpec((C, 1), lambda i: (0, 0)),
            pl.BlockSpec((1, C), lambda i: (0, 0)),
            pl.BlockSpec((C, 1), lambda i: (0, 0)),
        ],
        out_specs=pl.BlockSpec((GROUP, K, V), lambda i: (i, 0, 0)),
        out_shape=jax.ShapeDtypeStruct((BATCH, K, V), jnp.float32),
    )(x, W, b2, a1, a2)
    h = pl.pallas_call(
        _softmax0_kernel,
        out_shape=jax.ShapeDtypeStruct((BATCH, K * V), jnp.float32),
    )(s.reshape(BATCH, K * V))
    return jnp.transpose(h.reshape(BATCH, K, V), (0, 2, 1))
